# Initial kernel scaffold; baseline (speedup 1.0000x reference)
import jax
import jax.numpy as jnp
from jax.experimental import pallas as pl


def kernel(initial_features, relation_embeddings, W1, g1, b1, W2, g2, b2, edge_index, edge_type):
    raise NotImplementedError("write your pallas kernel here")


# speedup vs baseline: 2.8427x; 2.8427x over previous
import functools

import jax
import jax.numpy as jnp
from jax import lax
from jax.experimental import pallas as pl
from jax.experimental.pallas import tpu as pltpu
from jax.experimental.pallas import tpu_sc as plsc

D = 128
NC, NS = 2, 16
NW = NC * NS
K = 80


def _sc_kernel_body(do_deg, n_pad, n_edges, rp,
                    x_hbm, rel_hbm, src_hbm, tgt_hbm, typ_hbm, z_hbm,
                    out_hbm, hist_hbm, dega_hbm, degb_hbm,
                    acc_sh, rel_sh,
                    idxs, idxt, idxy, xrows, relrows, deg_local,
                    strip_buf, strip_acc,
                    sem0, sem1):
    c = lax.axis_index("c")
    s = lax.axis_index("s")
    ept = n_edges // NW
    rows_pt = n_pad // NS
    rel_pt = rp // NS
    base = (c * NS + s) * ept
    zv = jnp.zeros((16,), jnp.float32)

    pltpu.sync_copy(z_hbm, acc_sh.at[pl.ds(s * rows_pt, rows_pt)])
    pltpu.sync_copy(rel_hbm.at[pl.ds(s * rel_pt, rel_pt)],
                    rel_sh.at[pl.ds(s * rel_pt, rel_pt)])
    if do_deg:
        def zdeg(i, carry):
            deg_local[pl.ds(16 * i, 16)] = zv
            return carry
        lax.fori_loop(0, n_pad // 16, zdeg, 0)
    plsc.subcore_barrier()

    def chunk_body(ci, carry):
        eb = base + ci * K
        pltpu.sync_copy(src_hbm.at[pl.ds(eb, K)], idxs)
        pltpu.sync_copy(typ_hbm.at[pl.ds(eb, K)], idxy)
        pltpu.sync_copy(tgt_hbm.at[pl.ds(eb, K)], idxt)
        pltpu.async_copy(x_hbm.at[idxs], xrows, sem0).wait()
        pltpu.async_copy(rel_sh.at[idxy], relrows, sem1).wait()

        def edge_body(e, ecarry):
            xr = [xrows[e, pl.ds(16 * j, 16)] for j in range(8)]
            rr = [relrows[e, pl.ds(16 * j, 16)] for j in range(8)]
            p0 = xr[0] * rr[0] + xr[1] * rr[1]
            p1 = xr[2] * rr[2] + xr[3] * rr[3]
            p2 = xr[4] * rr[4] + xr[5] * rr[5]
            p3 = xr[6] * rr[6] + xr[7] * rr[7]
            t = jnp.float32(2.0) * jnp.sum((p0 + p1) + (p2 + p3))
            for j in range(8):
                xrows[e, pl.ds(16 * j, 16)] = xr[j] - t * rr[j]
            return ecarry

        lax.fori_loop(0, K, edge_body, 0)

        if do_deg:
            def deg_body(v, dcarry):
                tv = idxt[pl.ds(16 * v, 16)]
                cnt, lm = plsc.scan_count(tv)
                plsc.addupdate_scatter(deg_local, [tv],
                                       cnt.astype(jnp.float32), mask=lm)
                return dcarry
            lax.fori_loop(0, K // 16, deg_body, 0)

        pltpu.sync_copy(xrows, acc_sh.at[idxt], add=True)
        return carry

    lax.fori_loop(0, ept // K, chunk_body, 0)

    if do_deg:
        pltpu.sync_copy(deg_local,
                        hist_hbm.at[pl.ds((c * NS + s) * n_pad, n_pad)])
    plsc.subcore_barrier()

    pltpu.sync_copy(acc_sh.at[pl.ds(s * rows_pt, rows_pt)],
                    out_hbm.at[c, pl.ds(s * rows_pt, rows_pt)])

    if do_deg:
        def zstrip(i, carry):
            strip_acc[pl.ds(16 * i, 16)] = zv
            return carry
        lax.fori_loop(0, rows_pt // 16, zstrip, 0)

        def row_body(r, carry):
            pltpu.sync_copy(
                hist_hbm.at[pl.ds((c * NS + r) * n_pad + s * rows_pt,
                                  rows_pt)],
                strip_buf)
            def add_body(j, acarry):
                strip_acc[pl.ds(16 * j, 16)] = (strip_acc[pl.ds(16 * j, 16)]
                                                + strip_buf[pl.ds(16 * j, 16)])
                return acarry
            lax.fori_loop(0, rows_pt // 16, add_body, 0)
            return carry
        lax.fori_loop(0, NS, row_body, 0)

        @pl.when(c == 0)
        def _():
            pltpu.sync_copy(strip_acc, dega_hbm.at[pl.ds(s * rows_pt, rows_pt)])

        @pl.when(c == 1)
        def _():
            pltpu.sync_copy(strip_acc, degb_hbm.at[pl.ds(s * rows_pt, rows_pt)])


def _sc_scatter(x, reln, src, tgt, typ, zrows, n_pad, do_deg):
    n_edges = src.shape[0]
    rp = reln.shape[0]
    mesh = plsc.VectorSubcoreMesh(core_axis_name="c", subcore_axis_name="s",
                                  num_cores=NC, num_subcores=NS)
    body = functools.partial(_sc_kernel_body, do_deg, n_pad, n_edges, rp)
    return pl.kernel(
        body,
        out_type=(jax.ShapeDtypeStruct((NC, n_pad, D), jnp.float32),
                  jax.ShapeDtypeStruct((NW * n_pad,), jnp.float32),
                  jax.ShapeDtypeStruct((n_pad,), jnp.float32),
                  jax.ShapeDtypeStruct((n_pad,), jnp.float32)),
        mesh=mesh,
        compiler_params=pltpu.CompilerParams(needs_layout_passes=False),
        scratch_types=[
            pltpu.VMEM_SHARED((n_pad, D), jnp.float32),
            pltpu.VMEM_SHARED((rp, D), jnp.float32),
            pltpu.VMEM((K,), jnp.int32),
            pltpu.VMEM((K,), jnp.int32),
            pltpu.VMEM((K,), jnp.int32),
            pltpu.VMEM((K, D), jnp.float32),
            pltpu.VMEM((K, D), jnp.float32),
            pltpu.VMEM((n_pad,), jnp.float32),
            pltpu.VMEM((n_pad // NS,), jnp.float32),
            pltpu.VMEM((n_pad // NS,), jnp.float32),
            pltpu.SemaphoreType.DMA,
            pltpu.SemaphoreType.DMA,
        ],
        name="rgcn_sc_scatter",
    )(x, reln, src, tgt, typ, zrows)


def _norm_body(rel_ref, out_ref):
    r = rel_ref[...]
    nrm = jnp.sqrt(jnp.sum(r * r, axis=1, keepdims=True))
    out_ref[...] = r / jnp.maximum(nrm, 1e-6)


def _rel_normalize(relp):
    return pl.pallas_call(
        _norm_body,
        out_shape=jax.ShapeDtypeStruct(relp.shape, jnp.float32),
    )(relp)


def _dense_body(relu, pa_ref, pb_ref, da_ref, db_ref, xp_ref, w_ref, g_ref,
                b_ref, out_ref):
    sums = pa_ref[...] + pb_ref[...]
    deg = da_ref[...] + db_ref[...]
    avg = sums / jnp.maximum(deg, 1.0)
    h = lax.dot_general(avg, w_ref[...], (((1,), (1,)), ((), ())),
                        preferred_element_type=jnp.float32)
    h = h + xp_ref[...]
    mu = jnp.mean(h, axis=1, keepdims=True)
    var = jnp.mean((h - mu) ** 2, axis=1, keepdims=True)
    y = (h - mu) / jnp.sqrt(var + 1e-5) * g_ref[...] + b_ref[...]
    if relu:
        y = jnp.maximum(y, 0.0)
    out_ref[...] = y


def _dense_layer(pa, pb, da, db, xprev, w, g, b, relu):
    n_nodes = xprev.shape[0]
    bn = 2000
    grid = n_nodes // bn
    return pl.pallas_call(
        functools.partial(_dense_body, relu),
        out_shape=jax.ShapeDtypeStruct((n_nodes, D), jnp.float32),
        grid=(grid,),
        in_specs=[
            pl.BlockSpec((bn, D), lambda i: (i, 0)),
            pl.BlockSpec((bn, D), lambda i: (i, 0)),
            pl.BlockSpec((bn, 1), lambda i: (i, 0)),
            pl.BlockSpec((bn, 1), lambda i: (i, 0)),
            pl.BlockSpec((bn, D), lambda i: (i, 0)),
            pl.BlockSpec((D, D), lambda i: (0, 0)),
            pl.BlockSpec((1, D), lambda i: (0, 0)),
            pl.BlockSpec((1, D), lambda i: (0, 0)),
        ],
        out_specs=pl.BlockSpec((bn, D), lambda i: (i, 0)),
    )(pa, pb, da, db, xprev, w, g, b)


def kernel(initial_features, relation_embeddings, W1, g1, b1, W2, g2, b2,
           edge_index, edge_type):
    n_nodes = initial_features.shape[0]
    n_pad = ((n_nodes + 16 * NS - 1) // (16 * NS)) * (16 * NS)
    r = relation_embeddings.shape[0]
    rp = ((r + 8 * NS - 1) // (8 * NS)) * (8 * NS)
    src = edge_index[0].astype(jnp.int32)
    tgt = edge_index[1].astype(jnp.int32)
    typ = edge_type.astype(jnp.int32)
    relp = jnp.pad(relation_embeddings, ((0, rp - r), (0, 0)))
    reln = _rel_normalize(relp)
    zrows = jnp.zeros((n_pad // NS, D), jnp.float32)

    g1r = g1.reshape(1, D)
    b1r = b1.reshape(1, D)
    g2r = g2.reshape(1, D)
    b2r = b2.reshape(1, D)

    p1, _, dga, dgb = _sc_scatter(initial_features, reln, src, tgt, typ,
                                  zrows, n_pad, True)
    da = dga[:n_nodes].reshape(n_nodes, 1)
    db = dgb[:n_nodes].reshape(n_nodes, 1)
    x1 = _dense_layer(p1[0], p1[1], da, db, initial_features, W1, g1r, b1r,
                      True)
    p2, _, _, _ = _sc_scatter(x1, reln, src, tgt, typ, zrows, n_pad, False)
    x2 = _dense_layer(p2[0], p2[1], da, db, x1, W2, g2r, b2r, False)
    return x2


# speedup vs baseline: 4.5170x; 1.5890x over previous
import functools

import jax
import jax.numpy as jnp
from jax import lax
from jax.experimental import pallas as pl
from jax.experimental.pallas import tpu as pltpu
from jax.experimental.pallas import tpu_sc as plsc

D = 128
NC, NS = 2, 16
NW = NC * NS
K = 48
EGRP = NW * 288


def _sc_kernel_body(do_deg, n_pad, n_edges, rp,
                    x_hbm, rel_hbm, src_hbm, tgt_hbm, typ_hbm, z_hbm,
                    out_hbm, hist_hbm, dega_hbm, degb_hbm,
                    acc_sh, rel_sh,
                    idxs, idxt, idxy, xrows, relrows, deg_local,
                    strip_buf, strip_acc,
                    sem0, sem1, sem2):
    c = lax.axis_index("c")
    s = lax.axis_index("s")
    ept = n_edges // NW
    rows_pt = n_pad // NS
    rel_pt = rp // NS
    base = (c * NS + s) * ept
    zv = jnp.zeros((16,), jnp.float32)

    pltpu.sync_copy(z_hbm, acc_sh.at[pl.ds(s * rows_pt, rows_pt)])
    if do_deg:
        def zdeg(i, carry):
            deg_local[pl.ds(16 * i, 16)] = zv
            return carry
        lax.fori_loop(0, n_pad // 16, zdeg, 0)
    plsc.subcore_barrier()

    def load_idx(ch, b):
        eb = base + ch * K
        pltpu.async_copy(src_hbm.at[pl.ds(eb, K)], idxs[b], sem0[b])
        pltpu.async_copy(typ_hbm.at[pl.ds(eb, K)], idxy[b], sem0[b])
        pltpu.async_copy(tgt_hbm.at[pl.ds(eb, K)], idxt[b], sem0[b])
        pltpu.make_async_copy(src_hbm.at[pl.ds(eb, K)], idxs[b], sem0[b]).wait()
        pltpu.make_async_copy(typ_hbm.at[pl.ds(eb, K)], idxy[b], sem0[b]).wait()
        pltpu.make_async_copy(tgt_hbm.at[pl.ds(eb, K)], idxt[b], sem0[b]).wait()

    def issue_gather(b):
        pltpu.async_copy(x_hbm.at[idxs[b]], xrows[b], sem1[b])
        pltpu.async_copy(rel_hbm.at[idxy[b]], relrows[b], sem1[b])

    def wait_gather(b):
        pltpu.make_async_copy(x_hbm.at[idxs[b]], xrows[b], sem1[b]).wait()
        pltpu.make_async_copy(rel_hbm.at[idxy[b]], relrows[b], sem1[b]).wait()

    def compute(b):
        xb = xrows[b]
        rb = relrows[b]

        @plsc.parallel_loop(0, K, 1, unroll=2)
        def edge_body(e):
            xr = [xb[e, pl.ds(16 * j, 16)] for j in range(8)]
            rr = [rb[e, pl.ds(16 * j, 16)] for j in range(8)]
            p0 = xr[0] * rr[0] + xr[1] * rr[1]
            p1 = xr[2] * rr[2] + xr[3] * rr[3]
            p2 = xr[4] * rr[4] + xr[5] * rr[5]
            p3 = xr[6] * rr[6] + xr[7] * rr[7]
            t = jnp.float32(2.0) * jnp.sum((p0 + p1) + (p2 + p3))
            for j in range(8):
                xb[e, pl.ds(16 * j, 16)] = xr[j] - t * rr[j]

        if do_deg:
            def deg_body(v, dcarry):
                tv = idxt[b][pl.ds(16 * v, 16)]
                cnt, lm = plsc.scan_count(tv)
                plsc.addupdate_scatter(deg_local, [tv],
                                       cnt.astype(jnp.float32), mask=lm)
                return dcarry
            lax.fori_loop(0, K // 16, deg_body, 0)

    def issue_scatter(b):
        pltpu.async_copy(xrows[b], acc_sh.at[idxt[b]], sem2[b], add=True)

    def wait_scatter(b):
        pltpu.make_async_copy(xrows[b], acc_sh.at[idxt[b]], sem2[b]).wait()

    nch = ept // K
    load_idx(0, 0)
    issue_gather(0)
    wait_gather(0)
    load_idx(1, 1)
    issue_gather(1)
    compute(0)
    issue_scatter(0)

    def body(c, a):
        b = 1 - a
        wait_gather(a)
        wait_scatter(b)
        load_idx(c + 1, b)
        issue_gather(b)
        compute(a)
        issue_scatter(a)

    def pair_body(t, carry):
        body(2 * t + 1, 1)
        body(2 * t + 2, 0)
        return carry

    lax.fori_loop(0, (nch - 2) // 2, pair_body, 0)

    a = (nch - 1) % 2
    wait_gather(a)
    wait_scatter(1 - a)
    compute(a)
    pltpu.sync_copy(xrows[a], acc_sh.at[idxt[a]], add=True)

    if do_deg:
        pltpu.sync_copy(deg_local,
                        hist_hbm.at[pl.ds((c * NS + s) * n_pad, n_pad)])
    plsc.subcore_barrier()

    pltpu.sync_copy(acc_sh.at[pl.ds(s * rows_pt, rows_pt)],
                    out_hbm.at[c, pl.ds(s * rows_pt, rows_pt)])

    if do_deg:
        def zstrip(i, carry):
            strip_acc[pl.ds(16 * i, 16)] = zv
            return carry
        lax.fori_loop(0, rows_pt // 16, zstrip, 0)

        def row_body(r, carry):
            pltpu.sync_copy(
                hist_hbm.at[pl.ds((c * NS + r) * n_pad + s * rows_pt,
                                  rows_pt)],
                strip_buf)
            def add_body(j, acarry):
                strip_acc[pl.ds(16 * j, 16)] = (strip_acc[pl.ds(16 * j, 16)]
                                                + strip_buf[pl.ds(16 * j, 16)])
                return acarry
            lax.fori_loop(0, rows_pt // 16, add_body, 0)
            return carry
        lax.fori_loop(0, NS, row_body, 0)

        @pl.when(c == 0)
        def _():
            pltpu.sync_copy(strip_acc, dega_hbm.at[pl.ds(s * rows_pt, rows_pt)])

        @pl.when(c == 1)
        def _():
            pltpu.sync_copy(strip_acc, degb_hbm.at[pl.ds(s * rows_pt, rows_pt)])


def _sc_scatter(x, reln, src, tgt, typ, zrows, n_pad, do_deg):
    n_edges = src.shape[0]
    rp = reln.shape[0]
    mesh = plsc.VectorSubcoreMesh(core_axis_name="c", subcore_axis_name="s",
                                  num_cores=NC, num_subcores=NS)
    body = functools.partial(_sc_kernel_body, do_deg, n_pad, n_edges, rp)
    return pl.kernel(
        body,
        out_type=(jax.ShapeDtypeStruct((NC, n_pad, D), jnp.float32),
                  jax.ShapeDtypeStruct((NW * n_pad,), jnp.float32),
                  jax.ShapeDtypeStruct((n_pad,), jnp.float32),
                  jax.ShapeDtypeStruct((n_pad,), jnp.float32)),
        mesh=mesh,
        compiler_params=pltpu.CompilerParams(needs_layout_passes=False),
        scratch_types=[
            pltpu.VMEM_SHARED((n_pad, D), jnp.float32),
            pltpu.VMEM_SHARED((rp, D), jnp.float32),
            [pltpu.VMEM((K,), jnp.int32) for _ in range(3)],
            [pltpu.VMEM((K,), jnp.int32) for _ in range(3)],
            [pltpu.VMEM((K,), jnp.int32) for _ in range(3)],
            [pltpu.VMEM((K, D), jnp.float32) for _ in range(2)],
            [pltpu.VMEM((K, D), jnp.float32) for _ in range(2)],
            pltpu.VMEM((n_pad,), jnp.float32),
            pltpu.VMEM((n_pad // NS,), jnp.float32),
            pltpu.VMEM((n_pad // NS,), jnp.float32),
            [pltpu.SemaphoreType.DMA for _ in range(3)],
            [pltpu.SemaphoreType.DMA for _ in range(3)],
            [pltpu.SemaphoreType.DMA for _ in range(2)],
        ],
        name="rgcn_sc_scatter",
    )(x, reln, src, tgt, typ, zrows)


def _norm_body(rel_ref, out_ref):
    r = rel_ref[...]
    nrm = jnp.sqrt(jnp.sum(r * r, axis=1, keepdims=True))
    out_ref[...] = r / jnp.maximum(nrm, 1e-6)


def _rel_normalize(relp):
    return pl.pallas_call(
        _norm_body,
        out_shape=jax.ShapeDtypeStruct(relp.shape, jnp.float32),
    )(relp)


def _dense_body(relu, pa_ref, pb_ref, da_ref, db_ref, xp_ref, w_ref, g_ref,
                b_ref, out_ref):
    sums = pa_ref[...] + pb_ref[...]
    deg = da_ref[...] + db_ref[...]
    avg = sums / jnp.maximum(deg, 1.0)
    h = lax.dot_general(avg, w_ref[...], (((1,), (1,)), ((), ())),
                        preferred_element_type=jnp.float32)
    h = h + xp_ref[...]
    mu = jnp.mean(h, axis=1, keepdims=True)
    var = jnp.mean((h - mu) ** 2, axis=1, keepdims=True)
    y = (h - mu) / jnp.sqrt(var + 1e-5) * g_ref[...] + b_ref[...]
    if relu:
        y = jnp.maximum(y, 0.0)
    out_ref[...] = y


def _dense_layer(pa, pb, da, db, xprev, w, g, b, relu):
    n_nodes = xprev.shape[0]
    bn = 2000
    grid = n_nodes // bn
    return pl.pallas_call(
        functools.partial(_dense_body, relu),
        out_shape=jax.ShapeDtypeStruct((n_nodes, D), jnp.float32),
        grid=(grid,),
        in_specs=[
            pl.BlockSpec((bn, D), lambda i: (i, 0)),
            pl.BlockSpec((bn, D), lambda i: (i, 0)),
            pl.BlockSpec((bn, 1), lambda i: (i, 0)),
            pl.BlockSpec((bn, 1), lambda i: (i, 0)),
            pl.BlockSpec((bn, D), lambda i: (i, 0)),
            pl.BlockSpec((D, D), lambda i: (0, 0)),
            pl.BlockSpec((1, D), lambda i: (0, 0)),
            pl.BlockSpec((1, D), lambda i: (0, 0)),
        ],
        out_specs=pl.BlockSpec((bn, D), lambda i: (i, 0)),
    )(pa, pb, da, db, xprev, w, g, b)


def kernel(initial_features, relation_embeddings, W1, g1, b1, W2, g2, b2,
           edge_index, edge_type):
    n_nodes = initial_features.shape[0]
    n_pad = ((n_nodes + 16 * NS - 1) // (16 * NS)) * (16 * NS)
    r = relation_embeddings.shape[0]
    rp = ((r + 8 * NS - 1) // (8 * NS)) * (8 * NS)
    n_edges = edge_index.shape[1]
    e_pad = ((n_edges + EGRP - 1) // EGRP) * EGRP
    src = edge_index[0].astype(jnp.int32)
    tgt = edge_index[1].astype(jnp.int32)
    typ = edge_type.astype(jnp.int32)
    npad_e = e_pad - n_edges
    if npad_e:
        src = jnp.concatenate([src, jnp.zeros((npad_e,), jnp.int32)])
        tgt = jnp.concatenate([tgt,
                               jnp.full((npad_e,), n_nodes, jnp.int32)])
        typ = jnp.concatenate([typ, jnp.zeros((npad_e,), jnp.int32)])
    relp = jnp.pad(relation_embeddings, ((0, rp - r), (0, 0)))
    reln = _rel_normalize(relp)
    zrows = jnp.zeros((n_pad // NS, D), jnp.float32)

    g1r = g1.reshape(1, D)
    b1r = b1.reshape(1, D)
    g2r = g2.reshape(1, D)
    b2r = b2.reshape(1, D)

    p1, _, dga, dgb = _sc_scatter(initial_features, reln, src, tgt, typ,
                                  zrows, n_pad, True)
    da = dga[:n_nodes].reshape(n_nodes, 1)
    db = dgb[:n_nodes].reshape(n_nodes, 1)
    x1 = _dense_layer(p1[0], p1[1], da, db, initial_features, W1, g1r, b1r,
                      True)
    p2, _, _, _ = _sc_scatter(x1, reln, src, tgt, typ, zrows, n_pad, False)
    x2 = _dense_layer(p2[0], p2[1], da, db, x1, W2, g2r, b2r, False)
    return x2
